# Initial kernel scaffold; baseline (speedup 1.0000x reference)
import jax
import jax.numpy as jnp
from jax.experimental import pallas as pl


def kernel(xyz1, xyz2, features1, features2, W1, b1, gamma1, beta1, W2, b2, gamma2, beta2):
    raise NotImplementedError("write your pallas kernel here")


# speedup vs baseline: 21.2991x; 21.2991x over previous
import functools

import jax
import jax.numpy as jnp
from jax.experimental import pallas as pl

B, N, M = 4, 8192, 2048
C1, C2 = 16, 32
CIN = C1 + C2
H = 64
NB = 256
NBLK = N // NB
CNT = float(B * N)
HIGH = jax.lax.Precision.HIGHEST


def _stage_a(xyz1_ref, xyz2t_ref, f1_ref, f2_ref, w1t_ref, b1_ref,
             y1_ref, st_ref):
    b = pl.program_id(0)
    i = pl.program_id(1)

    @pl.when(jnp.logical_and(b == 0, i == 0))
    def _():
        st_ref[...] = jnp.zeros_like(st_ref)

    q = xyz1_ref[0]
    kt = xyz2t_ref[0]
    cross = jnp.dot(q.astype(jnp.bfloat16), kt.astype(jnp.bfloat16),
                    preferred_element_type=jnp.float32)
    qn = (q[:, 0:1] * q[:, 0:1] + q[:, 1:2] * q[:, 1:2]
          + q[:, 2:3] * q[:, 2:3])
    kn = (kt[0:1, :] * kt[0:1, :] + kt[1:2, :] * kt[1:2, :]
          + kt[2:3, :] * kt[2:3, :])
    d2 = qn + kn - 2.0 * cross
    d2 = jnp.maximum(d2, 0.0)

    iota = jax.lax.broadcasted_iota(jnp.int32, (NB, M), 1)
    dcur = d2
    aks = []
    dmins = []
    for _k in range(3):
        mk = jnp.min(dcur, axis=1, keepdims=True)
        ak = jnp.min(jnp.where(dcur == mk, iota, M), axis=1, keepdims=True)
        aks.append(ak)
        dmins.append(mk)
        dcur = jnp.where(iota == ak, jnp.float32(jnp.inf), dcur)

    winv = [1.0 / (jnp.sqrt(mk) + 1e-10) for mk in dmins]
    z = winv[0] + winv[1] + winv[2]
    s = None
    for k in range(3):
        part = jnp.where(iota == aks[k], winv[k] / z, 0.0)
        s = part if s is None else s + part

    f2 = f2_ref[0]
    interp = jnp.dot(s, f2, preferred_element_type=jnp.float32,
                     precision=HIGH)
    x = jnp.concatenate([interp, f1_ref[0]], axis=1)
    y1 = jnp.dot(x.astype(jnp.bfloat16), w1t_ref[...].astype(jnp.bfloat16),
                 preferred_element_type=jnp.float32) + b1_ref[...]
    y1_ref[0] = y1
    st_ref[0:1, :] += jnp.sum(y1, axis=0, keepdims=True)
    st_ref[1:2, :] += jnp.sum(y1 * y1, axis=0, keepdims=True)


def _stage_b(y1_ref, st1_ref, w2t_ref, b2_ref, g1_ref, be1_ref,
             y2_ref, st2_ref):
    b = pl.program_id(0)
    i = pl.program_id(1)

    @pl.when(jnp.logical_and(b == 0, i == 0))
    def _():
        st2_ref[...] = jnp.zeros_like(st2_ref)

    mu = st1_ref[0:1, :] / CNT
    var = st1_ref[1:2, :] / CNT - mu * mu
    rstd = jax.lax.rsqrt(var + 1e-5)
    y1 = y1_ref[0]
    zt = jnp.maximum((y1 - mu) * rstd * g1_ref[...] + be1_ref[...], 0.0)
    y2 = jnp.dot(zt.astype(jnp.bfloat16), w2t_ref[...].astype(jnp.bfloat16),
                 preferred_element_type=jnp.float32) + b2_ref[...]
    y2_ref[0] = y2
    st2_ref[0:1, :] += jnp.sum(y2, axis=0, keepdims=True)
    st2_ref[1:2, :] += jnp.sum(y2 * y2, axis=0, keepdims=True)


def _stage_c(y2_ref, st2_ref, g2_ref, be2_ref, out_ref):
    mu = st2_ref[0:1, :] / CNT
    var = st2_ref[1:2, :] / CNT - mu * mu
    rstd = jax.lax.rsqrt(var + 1e-5)
    y2 = y2_ref[0]
    out_ref[0] = jnp.maximum((y2 - mu) * rstd * g2_ref[...] + be2_ref[...],
                             0.0)


def kernel(xyz1, xyz2, features1, features2, W1, b1, gamma1, beta1,
           W2, b2, gamma2, beta2):
    xyz2t = jnp.transpose(xyz2, (0, 2, 1))
    w1t = jnp.transpose(W1)
    w2t = jnp.transpose(W2)
    b1r = b1[None, :]
    b2r = b2[None, :]
    g1r = gamma1[None, :]
    g2r = gamma2[None, :]
    be1r = beta1[None, :]
    be2r = beta2[None, :]

    grid = (B, NBLK)
    row_spec = lambda c: pl.BlockSpec((1, NB, c), lambda b, i: (b, i, 0))
    full2d = lambda shape: pl.BlockSpec(shape, lambda b, i: (0, 0))

    y1, st1 = pl.pallas_call(
        _stage_a,
        grid=grid,
        in_specs=[
            row_spec(3),
            pl.BlockSpec((1, 3, M), lambda b, i: (b, 0, 0)),
            row_spec(C1),
            pl.BlockSpec((1, M, C2), lambda b, i: (b, 0, 0)),
            full2d((CIN, H)),
            full2d((1, H)),
        ],
        out_specs=[row_spec(H), full2d((8, H))],
        out_shape=[
            jax.ShapeDtypeStruct((B, N, H), jnp.float32),
            jax.ShapeDtypeStruct((8, H), jnp.float32),
        ],
    )(xyz1, xyz2t, features1, features2, w1t, b1r)

    y2, st2 = pl.pallas_call(
        _stage_b,
        grid=grid,
        in_specs=[
            row_spec(H),
            full2d((8, H)),
            full2d((H, H)),
            full2d((1, H)),
            full2d((1, H)),
            full2d((1, H)),
        ],
        out_specs=[row_spec(H), full2d((8, H))],
        out_shape=[
            jax.ShapeDtypeStruct((B, N, H), jnp.float32),
            jax.ShapeDtypeStruct((8, H), jnp.float32),
        ],
    )(y1, st1, w2t, b2r, g1r, be1r)

    out = pl.pallas_call(
        _stage_c,
        grid=grid,
        in_specs=[
            row_spec(H),
            full2d((8, H)),
            full2d((1, H)),
            full2d((1, H)),
        ],
        out_specs=row_spec(H),
        out_shape=jax.ShapeDtypeStruct((B, N, H), jnp.float32),
    )(y2, st2, g2r, be2r)

    return out


# speedup vs baseline: 26.2169x; 1.2309x over previous
import functools

import jax
import jax.numpy as jnp
from jax import lax
from jax.experimental import pallas as pl
from jax.experimental.pallas import tpu as pltpu
from jax.experimental.pallas import tpu_sc as plsc

B, N, M = 4, 8192, 2048
C1, C2 = 16, 32
CIN = C1 + C2
H = 64
NB = 256
NBLK = N // NB
CNT = float(B * N)
HIGH = jax.lax.Precision.HIGHEST

NW = 32
PTS_W = B * N // NW


def _stage_a1(xyz1_ref, xyz2t_ref, gidx_ref, wn_ref):
    b = pl.program_id(0)
    q = xyz1_ref[0]
    kt = xyz2t_ref[0]
    cross = jnp.dot(q.astype(jnp.bfloat16), kt.astype(jnp.bfloat16),
                    preferred_element_type=jnp.float32)
    qn = (q[:, 0:1] * q[:, 0:1] + q[:, 1:2] * q[:, 1:2]
          + q[:, 2:3] * q[:, 2:3])
    kn = (kt[0:1, :] * kt[0:1, :] + kt[1:2, :] * kt[1:2, :]
          + kt[2:3, :] * kt[2:3, :])
    d2 = jnp.maximum(qn + kn - 2.0 * cross, 0.0)

    iota = jax.lax.broadcasted_iota(jnp.int32, (NB, M), 1)
    dcur = d2
    aks = []
    dmins = []
    for k in range(3):
        mk = jnp.min(dcur, axis=1, keepdims=True)
        ak = jnp.min(jnp.where(dcur == mk, iota, M), axis=1, keepdims=True)
        aks.append(ak)
        dmins.append(mk)
        if k < 2:
            dcur = jnp.where(iota == ak, jnp.float32(jnp.inf), dcur)

    zero = jnp.zeros_like(aks[0])
    gidx_ref[0] = jnp.concatenate(aks + [zero], axis=1)
    winv = [1.0 / (jnp.sqrt(mk) + 1e-10) for mk in dmins]
    z = winv[0] + winv[1] + winv[2]
    wn_ref[0] = jnp.concatenate(
        [wi / z for wi in winv] + [jnp.zeros_like(z)], axis=1)


def _interp_body(table_hbm, idx_hbm, w_hbm, out_hbm,
                 table_v, idx_v, w_v, out_v):
    wid = lax.axis_index("s") * 2 + lax.axis_index("c")
    batch = wid // (NW // B)
    pltpu.sync_copy(table_hbm.at[batch], table_v)
    pltpu.sync_copy(idx_hbm.at[pl.ds(wid * PTS_W * 4, PTS_W * 4)], idx_v)
    pltpu.sync_copy(w_hbm.at[pl.ds(wid * PTS_W * 4, PTS_W * 4)], w_v)

    def chunk(q, _):
        po = q * 4
        iv = idx_v[pl.ds(po * 4, 16)]
        wv = w_v[pl.ds(po * 4, 16)]
        for j in range(4):
            i0 = iv[4 * j] * C2
            i1 = iv[4 * j + 1] * C2
            i2 = iv[4 * j + 2] * C2
            w0 = wv[4 * j]
            w1 = wv[4 * j + 1]
            w2 = wv[4 * j + 2]
            lo = (w0 * table_v[pl.ds(i0, 16)]
                  + w1 * table_v[pl.ds(i1, 16)]
                  + w2 * table_v[pl.ds(i2, 16)])
            hi = (w0 * table_v[pl.ds(i0 + 16, 16)]
                  + w1 * table_v[pl.ds(i1 + 16, 16)]
                  + w2 * table_v[pl.ds(i2 + 16, 16)])
            out_v[pl.ds((po + j) * C2, 16)] = lo
            out_v[pl.ds((po + j) * C2 + 16, 16)] = hi
        return 0

    lax.fori_loop(0, PTS_W // 4, chunk, 0)
    pltpu.sync_copy(out_v, out_hbm.at[pl.ds(wid * PTS_W * C2, PTS_W * C2)])


def _sc_interp(table_flat, idx_flat, w_flat):
    mesh = plsc.VectorSubcoreMesh(core_axis_name="c", subcore_axis_name="s")
    f = functools.partial(
        pl.kernel, mesh=mesh,
        out_type=jax.ShapeDtypeStruct((B * N * C2,), jnp.float32),
        scratch_types=[
            pltpu.VMEM((M * C2,), jnp.float32),
            pltpu.VMEM((PTS_W * 4,), jnp.int32),
            pltpu.VMEM((PTS_W * 4,), jnp.float32),
            pltpu.VMEM((PTS_W * C2,), jnp.float32),
        ],
    )(_interp_body)
    return f(table_flat, idx_flat, w_flat)


def _stage_a2(g_ref, f1_ref, w1t_ref, b1_ref, y1_ref, st_ref):
    b = pl.program_id(0)
    i = pl.program_id(1)

    @pl.when(jnp.logical_and(b == 0, i == 0))
    def _():
        st_ref[...] = jnp.zeros_like(st_ref)

    interp = g_ref[0]
    x = jnp.concatenate([interp, f1_ref[0]], axis=1)
    y1 = jnp.dot(x.astype(jnp.bfloat16), w1t_ref[...].astype(jnp.bfloat16),
                 preferred_element_type=jnp.float32) + b1_ref[...]
    y1_ref[0] = y1
    st_ref[0:1, :] += jnp.sum(y1, axis=0, keepdims=True)
    st_ref[1:2, :] += jnp.sum(y1 * y1, axis=0, keepdims=True)


def _stage_b(y1_ref, st1_ref, w2t_ref, b2_ref, g1_ref, be1_ref,
             y2_ref, st2_ref):
    b = pl.program_id(0)
    i = pl.program_id(1)

    @pl.when(jnp.logical_and(b == 0, i == 0))
    def _():
        st2_ref[...] = jnp.zeros_like(st2_ref)

    mu = st1_ref[0:1, :] / CNT
    var = st1_ref[1:2, :] / CNT - mu * mu
    rstd = jax.lax.rsqrt(var + 1e-5)
    y1 = y1_ref[0]
    zt = jnp.maximum((y1 - mu) * rstd * g1_ref[...] + be1_ref[...], 0.0)
    y2 = jnp.dot(zt.astype(jnp.bfloat16), w2t_ref[...].astype(jnp.bfloat16),
                 preferred_element_type=jnp.float32) + b2_ref[...]
    y2_ref[0] = y2
    st2_ref[0:1, :] += jnp.sum(y2, axis=0, keepdims=True)
    st2_ref[1:2, :] += jnp.sum(y2 * y2, axis=0, keepdims=True)


def _stage_c(y2_ref, st2_ref, g2_ref, be2_ref, out_ref):
    mu = st2_ref[0:1, :] / CNT
    var = st2_ref[1:2, :] / CNT - mu * mu
    rstd = jax.lax.rsqrt(var + 1e-5)
    y2 = y2_ref[0]
    out_ref[0] = jnp.maximum((y2 - mu) * rstd * g2_ref[...] + be2_ref[...],
                             0.0)


def kernel(xyz1, xyz2, features1, features2, W1, b1, gamma1, beta1,
           W2, b2, gamma2, beta2):
    xyz2t = jnp.transpose(xyz2, (0, 2, 1))
    w1t = jnp.transpose(W1)
    w2t = jnp.transpose(W2)
    b1r = b1[None, :]
    b2r = b2[None, :]
    g1r = gamma1[None, :]
    g2r = gamma2[None, :]
    be1r = beta1[None, :]
    be2r = beta2[None, :]

    grid = (B, NBLK)
    row_spec = lambda c: pl.BlockSpec((1, NB, c), lambda b, i: (b, i, 0))
    full2d = lambda shape: pl.BlockSpec(shape, lambda b, i: (0, 0))

    gidx, wn = pl.pallas_call(
        _stage_a1,
        grid=grid,
        in_specs=[
            row_spec(3),
            pl.BlockSpec((1, 3, M), lambda b, i: (b, 0, 0)),
        ],
        out_specs=[row_spec(4), row_spec(4)],
        out_shape=[
            jax.ShapeDtypeStruct((B, N, 4), jnp.int32),
            jax.ShapeDtypeStruct((B, N, 4), jnp.float32),
        ],
    )(xyz1, xyz2t)

    interp = _sc_interp(features2.reshape(B, M * C2),
                        gidx.reshape(B * N * 4),
                        wn.reshape(B * N * 4))
    g3 = interp.reshape(B, N, C2)

    y1, st1 = pl.pallas_call(
        _stage_a2,
        grid=grid,
        in_specs=[
            row_spec(C2),
            row_spec(C1),
            full2d((CIN, H)),
            full2d((1, H)),
        ],
        out_specs=[row_spec(H), full2d((8, H))],
        out_shape=[
            jax.ShapeDtypeStruct((B, N, H), jnp.float32),
            jax.ShapeDtypeStruct((8, H), jnp.float32),
        ],
    )(g3, features1, w1t, b1r)

    y2, st2 = pl.pallas_call(
        _stage_b,
        grid=grid,
        in_specs=[
            row_spec(H),
            full2d((8, H)),
            full2d((H, H)),
            full2d((1, H)),
            full2d((1, H)),
            full2d((1, H)),
        ],
        out_specs=[row_spec(H), full2d((8, H))],
        out_shape=[
            jax.ShapeDtypeStruct((B, N, H), jnp.float32),
            jax.ShapeDtypeStruct((8, H), jnp.float32),
        ],
    )(y1, st1, w2t, b2r, g1r, be1r)

    out = pl.pallas_call(
        _stage_c,
        grid=grid,
        in_specs=[
            row_spec(H),
            full2d((8, H)),
            full2d((1, H)),
            full2d((1, H)),
        ],
        out_specs=row_spec(H),
        out_shape=jax.ShapeDtypeStruct((B, N, H), jnp.float32),
    )(y2, st2, g2r, be2r)

    return out


# speedup vs baseline: 45.3934x; 1.7315x over previous
import functools

import jax
import jax.numpy as jnp
from jax import lax
from jax.experimental import pallas as pl
from jax.experimental.pallas import tpu as pltpu
from jax.experimental.pallas import tpu_sc as plsc

B, N, M = 4, 8192, 2048
C1, C2 = 16, 32
CIN = C1 + C2
H = 64
NB = 1024
NBLK = N // NB
NP = 2048
NPBLK = N // NP
CNT = float(B * N)
HIGH = jax.lax.Precision.HIGHEST

NW = 32
PTS_W = B * N // NW


def _stage_a1(xyz1_ref, xyz2t_ref, gidx_ref, wn_ref):
    b = pl.program_id(0)
    q = xyz1_ref[0]
    kt = xyz2t_ref[0]
    cross = jnp.dot(q.astype(jnp.bfloat16), kt.astype(jnp.bfloat16),
                    preferred_element_type=jnp.float32)
    qn = (q[:, 0:1] * q[:, 0:1] + q[:, 1:2] * q[:, 1:2]
          + q[:, 2:3] * q[:, 2:3])
    kn = (kt[0:1, :] * kt[0:1, :] + kt[1:2, :] * kt[1:2, :]
          + kt[2:3, :] * kt[2:3, :])
    d2 = jnp.maximum(qn + kn - 2.0 * cross, 0.0)

    iota = jax.lax.broadcasted_iota(jnp.int32, (NB, M), 1).astype(jnp.float32)
    dcur = d2
    aks = []
    dmins = []
    for k in range(3):
        mk = jnp.min(dcur, axis=1, keepdims=True)
        ak = jnp.min(jnp.where(dcur == mk, iota, jnp.float32(M)),
                     axis=1, keepdims=True)
        aks.append(ak)
        dmins.append(mk)
        if k < 2:
            dcur = jnp.where(iota == ak, jnp.float32(jnp.inf), dcur)

    zero = jnp.zeros_like(aks[0])
    gidx_ref[0] = jnp.concatenate(aks + [zero], axis=1).astype(jnp.int32)
    winv = [1.0 / (jnp.sqrt(mk) + 1e-10) for mk in dmins]
    z = winv[0] + winv[1] + winv[2]
    wn_ref[0] = jnp.concatenate(
        [wi / z for wi in winv] + [jnp.zeros_like(z)], axis=1)


def _interp_body(table_hbm, idx_hbm, w_hbm, out_hbm,
                 table_v, idx_v, w_v, out_v):
    wid = lax.axis_index("s") * 2 + lax.axis_index("c")
    batch = wid // (NW // B)
    pltpu.sync_copy(table_hbm.at[batch], table_v)
    pltpu.sync_copy(idx_hbm.at[pl.ds(wid * PTS_W * 4, PTS_W * 4)], idx_v)
    pltpu.sync_copy(w_hbm.at[pl.ds(wid * PTS_W * 4, PTS_W * 4)], w_v)

    def chunk(q, _):
        po = q * 4
        iv = idx_v[pl.ds(po * 4, 16)]
        wv = w_v[pl.ds(po * 4, 16)]
        for j in range(4):
            i0 = iv[4 * j] * C2
            i1 = iv[4 * j + 1] * C2
            i2 = iv[4 * j + 2] * C2
            w0 = wv[4 * j]
            w1 = wv[4 * j + 1]
            w2 = wv[4 * j + 2]
            lo = (w0 * table_v[pl.ds(i0, 16)]
                  + w1 * table_v[pl.ds(i1, 16)]
                  + w2 * table_v[pl.ds(i2, 16)])
            hi = (w0 * table_v[pl.ds(i0 + 16, 16)]
                  + w1 * table_v[pl.ds(i1 + 16, 16)]
                  + w2 * table_v[pl.ds(i2 + 16, 16)])
            out_v[pl.ds((po + j) * C2, 16)] = lo
            out_v[pl.ds((po + j) * C2 + 16, 16)] = hi
        return 0

    lax.fori_loop(0, PTS_W // 4, chunk, 0)
    pltpu.sync_copy(out_v, out_hbm.at[pl.ds(wid * PTS_W * C2, PTS_W * C2)])


def _sc_interp(table_flat, idx_flat, w_flat):
    mesh = plsc.VectorSubcoreMesh(core_axis_name="c", subcore_axis_name="s")
    f = functools.partial(
        pl.kernel, mesh=mesh,
        out_type=jax.ShapeDtypeStruct((B * N * C2,), jnp.float32),
        scratch_types=[
            pltpu.VMEM((M * C2,), jnp.float32),
            pltpu.VMEM((PTS_W * 4,), jnp.int32),
            pltpu.VMEM((PTS_W * 4,), jnp.float32),
            pltpu.VMEM((PTS_W * C2,), jnp.float32),
        ],
    )(_interp_body)
    return f(table_flat, idx_flat, w_flat)


def _mlp(g_ref, f1_ref, w1t_ref, b1_ref, w2t_ref, b2_ref,
         g1_ref, be1_ref, g2_ref, be2_ref, out_ref, y_scr, st1, st2):
    p = pl.program_id(0)
    b = pl.program_id(1)
    i = pl.program_id(2)
    blk = b * NPBLK + i
    first = jnp.logical_and(b == 0, i == 0)

    @pl.when(jnp.logical_and(p == 0, first))
    def _():
        st1[...] = jnp.zeros_like(st1)

    @pl.when(jnp.logical_and(p == 1, first))
    def _():
        st2[...] = jnp.zeros_like(st2)

    @pl.when(p == 0)
    def _():
        x = jnp.concatenate([g_ref[0], f1_ref[0]], axis=1)
        y1 = jnp.dot(x.astype(jnp.bfloat16),
                     w1t_ref[...].astype(jnp.bfloat16),
                     preferred_element_type=jnp.float32) + b1_ref[...]
        y_scr[blk] = y1
        st1[0:1, :] += jnp.sum(y1, axis=0, keepdims=True)
        st1[1:2, :] += jnp.sum(y1 * y1, axis=0, keepdims=True)

    @pl.when(p == 1)
    def _():
        mu = st1[0:1, :] / CNT
        var = st1[1:2, :] / CNT - mu * mu
        rstd = jax.lax.rsqrt(var + 1e-5)
        y1 = y_scr[blk]
        zt = jnp.maximum((y1 - mu) * rstd * g1_ref[...] + be1_ref[...], 0.0)
        y2 = jnp.dot(zt.astype(jnp.bfloat16),
                     w2t_ref[...].astype(jnp.bfloat16),
                     preferred_element_type=jnp.float32) + b2_ref[...]
        y_scr[blk] = y2
        st2[0:1, :] += jnp.sum(y2, axis=0, keepdims=True)
        st2[1:2, :] += jnp.sum(y2 * y2, axis=0, keepdims=True)

    @pl.when(p == 2)
    def _():
        mu = st2[0:1, :] / CNT
        var = st2[1:2, :] / CNT - mu * mu
        rstd = jax.lax.rsqrt(var + 1e-5)
        y2 = y_scr[blk]
        out_ref[0] = jnp.maximum((y2 - mu) * rstd * g2_ref[...]
                                 + be2_ref[...], 0.0)


def kernel(xyz1, xyz2, features1, features2, W1, b1, gamma1, beta1,
           W2, b2, gamma2, beta2):
    xyz2t = jnp.transpose(xyz2, (0, 2, 1))
    w1t = jnp.transpose(W1)
    w2t = jnp.transpose(W2)
    b1r = b1[None, :]
    b2r = b2[None, :]
    g1r = gamma1[None, :]
    g2r = gamma2[None, :]
    be1r = beta1[None, :]
    be2r = beta2[None, :]

    grid = (B, NBLK)
    grid_p = (B, NPBLK)
    row_spec = lambda c: pl.BlockSpec((1, NB, c), lambda b, i: (b, i, 0))
    rowp_spec = lambda c: pl.BlockSpec((1, NP, c), lambda b, i: (b, i, 0))
    full2d = lambda shape: pl.BlockSpec(shape, lambda b, i: (0, 0))

    gidx, wn = pl.pallas_call(
        _stage_a1,
        grid=grid,
        in_specs=[
            row_spec(3),
            pl.BlockSpec((1, 3, M), lambda b, i: (b, 0, 0)),
        ],
        out_specs=[row_spec(4), row_spec(4)],
        out_shape=[
            jax.ShapeDtypeStruct((B, N, 4), jnp.int32),
            jax.ShapeDtypeStruct((B, N, 4), jnp.float32),
        ],
    )(xyz1, xyz2t)

    interp = _sc_interp(features2.reshape(B, M * C2),
                        gidx.reshape(B * N * 4),
                        wn.reshape(B * N * 4))
    g3 = interp.reshape(B, N, C2)

    def mlp_in(c):
        return pl.BlockSpec(
            (1, NP, c),
            lambda p, b, i: (jnp.where(p == 0, b, 0),
                             jnp.where(p == 0, i, 0), 0))

    small = lambda shape: pl.BlockSpec(shape, lambda p, b, i: (0, 0))
    out = pl.pallas_call(
        _mlp,
        grid=(3, B, NPBLK),
        in_specs=[
            mlp_in(C2),
            mlp_in(C1),
            small((CIN, H)),
            small((1, H)),
            small((H, H)),
            small((1, H)),
            small((1, H)),
            small((1, H)),
            small((1, H)),
            small((1, H)),
        ],
        out_specs=pl.BlockSpec(
            (1, NP, H),
            lambda p, b, i: (jnp.where(p == 2, b, 0),
                             jnp.where(p == 2, i, 0), 0)),
        out_shape=jax.ShapeDtypeStruct((B, N, H), jnp.float32),
        scratch_shapes=[
            pltpu.VMEM((B * NPBLK, NP, H), jnp.float32),
            pltpu.VMEM((8, H), jnp.float32),
            pltpu.VMEM((8, H), jnp.float32),
        ],
    )(g3, features1, w1t, b1r, w2t, b2r, g1r, be1r, g2r, be2r)

    return out
